# Initial kernel scaffold; baseline (speedup 1.0000x reference)
import jax
import jax.numpy as jnp
from jax.experimental import pallas as pl


def kernel(h, x, edges, vel, edge_attr, params):
    raise NotImplementedError("write your pallas kernel here")


# speedup vs baseline: 1.4145x; 1.4145x over previous
import functools

import jax
import jax.numpy as jnp
import numpy as np
from jax import lax
from jax.experimental import pallas as pl
from jax.experimental.pallas import tpu as pltpu
from jax.experimental.pallas import tpu_sc as plsc

F32 = jnp.float32
NPTS = 5
NW = 32
CH = 128
EB = 2048
BN = 2048
DGT = 256
DGS = 256
DA = 128


def _silu(v):
    return v * (1.0 / (1.0 + jnp.exp(-v)))


def _dot(a, b):
    return jnp.dot(a, b, preferred_element_type=F32)


@functools.lru_cache(maxsize=None)
def _make_gather2(EP, NP, D):
    n_ch = EP // NW // CH
    mesh = plsc.VectorSubcoreMesh(core_axis_name="c", subcore_axis_name="s")

    @functools.partial(
        pl.kernel,
        out_type=[jax.ShapeDtypeStruct((EP, D), F32),
                  jax.ShapeDtypeStruct((EP, D), F32)],
        mesh=mesh,
        scratch_types=[
            pltpu.VMEM((n_ch, CH), jnp.int32),
            pltpu.VMEM((n_ch, CH), jnp.int32),
            pltpu.VMEM((CH, D), F32),
            pltpu.VMEM((CH, D), F32),
            pltpu.SemaphoreType.DMA,
            pltpu.SemaphoreType.DMA,
        ],
    )
    def gk(tab_a, tab_b, ridx_h, cidx_h, out_a, out_b,
           ridx, cidx, buf_a, buf_b, sem_a, sem_b):
        c = lax.axis_index("c")
        s = lax.axis_index("s")
        w = s * 2 + c
        pltpu.sync_copy(ridx_h.at[pl.ds(w * n_ch, n_ch)], ridx)
        pltpu.sync_copy(cidx_h.at[pl.ds(w * n_ch, n_ch)], cidx)

        def body(j, carry):
            ca = pltpu.async_copy(tab_a.at[ridx.at[j]], buf_a, sem_a)
            cb = pltpu.async_copy(tab_b.at[cidx.at[j]], buf_b, sem_b)
            ca.wait()
            cb.wait()
            base = pl.multiple_of((w * n_ch + j) * CH, CH)
            pltpu.sync_copy(buf_a, out_a.at[pl.ds(base, CH)])
            pltpu.sync_copy(buf_b, out_b.at[pl.ds(base, CH)])
            return carry

        lax.fori_loop(0, n_ch, body, 0)

    return gk


@functools.lru_cache(maxsize=None)
def _make_scatter(EP, NP, D):
    n_ch = EP // 16 // CH
    rpt = NP // 16
    mesh = plsc.VectorSubcoreMesh(core_axis_name="c", subcore_axis_name="s")

    @functools.partial(
        pl.kernel,
        out_type=jax.ShapeDtypeStruct((2, NP, D), F32),
        mesh=mesh,
        scratch_types=[
            pltpu.VMEM((n_ch, CH), jnp.int32),
            pltpu.VMEM((CH, D), F32),
            pltpu.VMEM_SHARED((NP, D), F32),
            pltpu.SemaphoreType.DMA,
        ],
    )
    def sk(pay, ridx_h, zeros_h, out, ridx, buf, acc, sem):
        c = lax.axis_index("c")
        s = lax.axis_index("s")
        lane0 = pl.multiple_of(c * D, D)
        pltpu.sync_copy(zeros_h.at[pl.ds(s * rpt, rpt)],
                        acc.at[pl.ds(s * rpt, rpt)])
        pltpu.sync_copy(ridx_h.at[pl.ds(s * n_ch, n_ch)], ridx)
        plsc.subcore_barrier()

        def body(j, carry):
            base = pl.multiple_of((s * n_ch + j) * CH, CH)
            pltpu.sync_copy(pay.at[pl.ds(base, CH), pl.ds(lane0, D)], buf)
            pltpu.sync_copy(buf, acc.at[ridx.at[j]], add=True)
            return carry

        lax.fori_loop(0, n_ch, body, 0)
        plsc.subcore_barrier()
        pltpu.sync_copy(acc.at[pl.ds(s * rpt, rpt)],
                        out.at[c, pl.ds(s * rpt, rpt)])

    return sk


def _xprep_body(x15_ref, xc15_ref, cent15_ref):
    x15 = x15_ref[...]
    cents = []
    for cdim in range(3):
        acc = x15[:, cdim:cdim + 1]
        for p in range(1, NPTS):
            acc = acc + x15[:, 3 * p + cdim:3 * p + cdim + 1]
        cents.append(acc * (1.0 / NPTS))
    cent15 = jnp.concatenate(cents * NPTS, axis=1)
    cent15_ref[...] = cent15
    xc15_ref[...] = x15 - cent15


def _node_prep_body(h_ref, xp_ref, we_t, be, wr_t, wc_t, b0,
                    h0_ref, tab_a_ref, tab_b_ref):
    h0 = _dot(h_ref[...], we_t[...]) + be[...]
    xp = xp_ref[...]
    zp = jnp.zeros((h0.shape[0], DGT - 144), dtype=F32)
    h0_ref[...] = h0
    tab_a_ref[...] = jnp.concatenate([_dot(h0, wr_t[...]) + b0[...], xp, zp], axis=1)
    tab_b_ref[...] = jnp.concatenate([_dot(h0, wc_t[...]), xp, zp], axis=1)


def _geometry(hrx, hcx):
    a = [hrx[:, 128 + k:129 + k] for k in range(3)]
    b = [hcx[:, 128 + k:129 + k] for k in range(3)]
    cd = [a[k] - b[k] for k in range(3)]
    rad = cd[0] * cd[0] + cd[1] * cd[1] + cd[2] * cd[2]
    cc = [a[1] * b[2] - a[2] * b[1],
          a[2] * b[0] - a[0] * b[2],
          a[0] * b[1] - a[1] * b[0]]
    cv = [cd[1] * cc[2] - cd[2] * cc[1],
          cd[2] * cc[0] - cd[0] * cc[2],
          cd[0] * cc[1] - cd[1] * cc[0]]
    return a, b, cd, rad, cc, cv


def _edge_tail(hrow, hcol, ef, rad, cd, cc, cv,
               w_rad, wf_t, w1_t, b1, wc0_t, bc0, wc1_t):
    z = hrow + hcol + rad * w_rad[...] + _dot(ef, wf_t[...])
    m = _silu(_dot(_silu(z), w1_t[...]) + b1[...])
    c2 = _silu(_dot(m, wc0_t[...]) + bc0[...])
    coff = _dot(c2, wc1_t[...])
    co = [coff[:, k:k + 1] for k in range(3)]
    tr = [cd[k] * co[0] + cc[k] * co[1] + cv[k] * co[2] for k in range(3)]
    ones = jnp.ones_like(rad)
    zpad = jnp.zeros((m.shape[0], DGS - 132), dtype=F32)
    return jnp.concatenate([m, tr[0], tr[1], tr[2], ones, zpad], axis=1)


def _edge_mid_body(hrx_ref, hcx_ref, ef_ref,
                   w_rad, wf_t, w1_t, b1, wc0_t, bc0, wc1_t,
                   pay_ref):
    hrx = hrx_ref[...]
    hcx = hcx_ref[...]
    _, _, cd, rad, cc, cv = _geometry(hrx, hcx)
    pay_ref[...] = _edge_tail(hrx[:, :128], hcx[:, :128], ef_ref[...],
                              rad, cd, cc, cv,
                              w_rad, wf_t, w1_t, b1, wc0_t, bc0, wc1_t)


def _edge0_body(hrx_ref, hcx_ref, ea_ref,
                fw2pi, wcm_t, bcm, w_sin, w_cos, wei_t, wej_t, bp0,
                wp1_t, bp1, wp2_t, bp2, we0_t, be0, we1_t, be1,
                w_rad, wf_t, w1_t, b1, wc0_t, bc0, wc1_t,
                pay_ref, ef_ref):
    hrx = hrx_ref[...]
    hcx = hcx_ref[...]
    a, b, cd, rad, cc, cv = _geometry(hrx, hcx)

    inv_norm = 1.0 / (jnp.sqrt(rad) + 1.0)
    dn = [cd[k] * inv_norm for k in range(3)]
    ccn = jnp.sqrt(cc[0] * cc[0] + cc[1] * cc[1] + cc[2] * cc[2])
    inv_cn = 1.0 / (ccn + 1.0)
    cn = [cc[k] * inv_cn for k in range(3)]
    vt = [dn[1] * cn[2] - dn[2] * cn[1],
          dn[2] * cn[0] - dn[0] * cn[2],
          dn[0] * cn[1] - dn[1] * cn[0]]
    ci = [dn[0] * a[0] + dn[1] * a[1] + dn[2] * a[2],
          cn[0] * a[0] + cn[1] * a[1] + cn[2] * a[2],
          vt[0] * a[0] + vt[1] * a[1] + vt[2] * a[2]]
    cj = [dn[0] * b[0] + dn[1] * b[1] + dn[2] * b[2],
          cn[0] * b[0] + cn[1] * b[1] + cn[2] * b[2],
          vt[0] * b[0] + vt[1] * b[1] + vt[2] * b[2]]
    ni = jnp.sqrt(ci[0] * ci[0] + ci[1] * ci[1] + ci[2] * ci[2])
    nj = jnp.sqrt(cj[0] * cj[0] + cj[1] * cj[1] + cj[2] * cj[2])
    dotij = ci[0] * cj[0] + ci[1] * cj[1] + ci[2] * cj[2]
    pcos = dotij / (ni + 1e-05) / (nj + 1e-05)
    psin = jnp.sqrt(jnp.clip(1.0 - pcos * pcos, 1e-12, None))

    def femb(cvec):
        parts = []
        for k in range(3):
            xp = cvec[k] * fw2pi[...]
            parts.append(jnp.sin(xp))
            parts.append(jnp.cos(xp))
        return jnp.concatenate(parts, axis=1)

    ei = _dot(femb(ci), wcm_t[...]) + bcm[...]
    ej = _dot(femb(cj), wcm_t[...]) + bcm[...]

    e0 = _silu(psin * w_sin[...] + pcos * w_cos[...]
               + _dot(ei, wei_t[...]) + _dot(ej, wej_t[...]) + bp0[...])
    e1 = _silu(_dot(e0, wp1_t[...]) + bp1[...])
    eemb = _dot(e1, wp2_t[...]) + bp2[...]

    efa = _silu(_dot(ea_ref[...], we0_t[...]) + be0[...])
    efb = _silu(_dot(efa, we1_t[...]) + be1[...])
    ef = eemb * efb
    ef_ref[...] = ef

    pay_ref[...] = _edge_tail(hrx[:, :128], hcx[:, :128], ef,
                              rad, cd, cc, cv,
                              w_rad, wf_t, w1_t, b1, wc0_t, bc0, wc1_t)


def _node_update_core(h, xp, velp, acc,
                      wn0a_t, wn0b_t, bn0, wn1_t, bn1,
                      wv0_t, bv0, wv1_t, bv1):
    aggn = acc[0]
    aux = acc[1]
    cnt = jnp.maximum(aux[:, 3:4], 1.0)
    lane = lax.broadcasted_iota(jnp.int32, (1, 16), 1)
    mask = (lane < 3).astype(F32)
    agg16 = aux[:, :16] * (mask / cnt)
    vterm = _dot(_silu(_dot(h, wv0_t[...]) + bv0[...]), wv1_t[...]) + bv1[...]
    xn = xp + agg16 + vterm * velp
    hn = h + _dot(_silu(_dot(h, wn0a_t[...]) + _dot(aggn, wn0b_t[...])
                        + bn0[...]), wn1_t[...]) + bn1[...]
    return hn, xn


def _node_update_mid_body(h_ref, xp_ref, velp_ref, acc_ref,
                          wn0a_t, wn0b_t, bn0, wn1_t, bn1,
                          wv0_t, bv0, wv1_t, bv1,
                          wr_t, wc_t, b0,
                          hn_ref, xn_ref, tab_a_ref, tab_b_ref):
    hn, xn = _node_update_core(h_ref[...], xp_ref[...], velp_ref[...],
                               acc_ref[...], wn0a_t, wn0b_t, bn0, wn1_t, bn1,
                               wv0_t, bv0, wv1_t, bv1)
    hn_ref[...] = hn
    xn_ref[...] = xn
    zp = jnp.zeros((hn.shape[0], DGT - 144), dtype=F32)
    tab_a_ref[...] = jnp.concatenate([_dot(hn, wr_t[...]) + b0[...], xn, zp], axis=1)
    tab_b_ref[...] = jnp.concatenate([_dot(hn, wc_t[...]), xn, zp], axis=1)


def _node_update_last_body(h_ref, xp_ref, velp_ref, acc_ref,
                           wn0a_t, wn0b_t, bn0, wn1_t, bn1,
                           wv0_t, bv0, wv1_t, bv1,
                           hn_ref, xn_ref):
    hn, xn = _node_update_core(h_ref[...], xp_ref[...], velp_ref[...],
                               acc_ref[...], wn0a_t, wn0b_t, bn0, wn1_t, bn1,
                               wv0_t, bv0, wv1_t, bv1)
    hn_ref[...] = hn
    xn_ref[...] = xn


def _final_body(xf15_ref, cent15_ref, out15_ref):
    out15_ref[...] = xf15_ref[...] + cent15_ref[...]


def _rep(shape):
    return pl.BlockSpec(shape, lambda i: (0,) * len(shape))


def _t2(p):
    return p["W"].T, p["b"].reshape(1, -1)


def kernel(h, x, edges, vel, edge_attr, params):
    N, d_feat = h.shape
    E = edges.shape[1]
    G = N // NPTS
    HID = params["emb_node"]["W"].shape[0]
    align = NW * CH * 8
    EP = ((E + align - 1) // align) * align
    NP = ((N + 1 + BN - 1) // BN) * BN
    trash = N

    row = edges[0].astype(jnp.int32)
    col = edges[1].astype(jnp.int32)
    pad_i = jnp.full((EP - E,), trash, dtype=jnp.int32)
    rowp = jnp.concatenate([row, pad_i]).reshape(EP // CH, CH)
    colp = jnp.concatenate([col, pad_i]).reshape(EP // CH, CH)

    h_pad = jnp.pad(h, ((0, NP - N), (0, 0)))
    velp = jnp.pad(vel, ((0, NP - N), (0, 13)))
    ea_pad = jnp.pad(edge_attr, ((0, EP - E), (0, 0)))
    zeros_acc = jnp.zeros((NP, DA), dtype=F32)

    we_t, be = _t2(params["emb_node"])
    p0w = params["proj0"]["W"]
    w_sin = p0w[:, 0].reshape(1, HID)
    w_cos = p0w[:, 1].reshape(1, HID)
    wei_t = p0w[:, 2:2 + HID].T
    wej_t = p0w[:, 2 + HID:2 + 2 * HID].T
    bp0 = params["proj0"]["b"].reshape(1, HID)
    wp1_t, bp1 = _t2(params["proj1"])
    wp2_t, bp2 = _t2(params["proj2"])
    wcm_t, bcm = _t2(params["coff_mlp"])
    we0_t, be0 = _t2(params["emb_edge0"])
    we1_t, be1 = _t2(params["emb_edge1"])
    fw2pi = (params["fourier_W"] * (2.0 * np.pi)).reshape(1, -1)

    gcl = []
    for p in params["gcls"]:
        e0w = p["edge0"]["W"]
        gcl.append(dict(
            wr_t=e0w[:, :HID].T,
            wc_t=e0w[:, HID:2 * HID].T,
            w_rad=e0w[:, 2 * HID].reshape(1, HID),
            wf_t=e0w[:, 2 * HID + 1:].T,
            b0=p["edge0"]["b"].reshape(1, HID),
            w1_t=p["edge1"]["W"].T, b1=p["edge1"]["b"].reshape(1, HID),
            wc0_t=p["coord0"]["W"].T, bc0=p["coord0"]["b"].reshape(1, HID),
            wc1_t=p["coord1"]["W"].T,
            wn0a_t=p["node0"]["W"][:, :HID].T,
            wn0b_t=p["node0"]["W"][:, HID:].T,
            bn0=p["node0"]["b"].reshape(1, HID),
            wn1_t=p["node1"]["W"].T, bn1=p["node1"]["b"].reshape(1, HID),
            wv0_t=p["vel0"]["W"].T, bv0=p["vel0"]["b"].reshape(1, HID),
            wv1_t=p["vel1"]["W"].T, bv1=p["vel1"]["b"].reshape(1, 1),
        ))

    x15 = x.reshape(G, 3 * NPTS)
    xc15, cent15 = pl.pallas_call(
        _xprep_body,
        out_shape=[jax.ShapeDtypeStruct((G, 3 * NPTS), F32)] * 2,
    )(x15)
    x_pad = jnp.pad(xc15.reshape(N, 3), ((0, NP - N), (0, 13)))

    ng = NP // BN
    node_w = _rep((HID, HID))
    node_b = _rep((1, HID))
    h0, tab_a, tab_b = pl.pallas_call(
        _node_prep_body,
        grid=(ng,),
        in_specs=[pl.BlockSpec((BN, d_feat), lambda i: (i, 0)),
                  pl.BlockSpec((BN, 16), lambda i: (i, 0)),
                  _rep((d_feat, HID)), node_b, node_w, node_w, node_b],
        out_specs=[pl.BlockSpec((BN, HID), lambda i: (i, 0)),
                   pl.BlockSpec((BN, DGT), lambda i: (i, 0)),
                   pl.BlockSpec((BN, DGT), lambda i: (i, 0))],
        out_shape=[jax.ShapeDtypeStruct((NP, HID), F32),
                   jax.ShapeDtypeStruct((NP, DGT), F32),
                   jax.ShapeDtypeStruct((NP, DGT), F32)],
    )(h_pad, x_pad, we_t, be, gcl[0]["wr_t"], gcl[0]["wc_t"], gcl[0]["b0"])

    gather2 = _make_gather2(EP, NP, DGT)
    scatter = _make_scatter(EP, NP, DA)

    eg = EP // EB
    eb_feat = pl.BlockSpec((EB, HID), lambda i: (i, 0))
    eb_full = pl.BlockSpec((EB, DGT), lambda i: (i, 0))
    eb_pay = pl.BlockSpec((EB, DGS), lambda i: (i, 0))

    def edge_weight_specs():
        return [_rep((1, HID)), _rep((HID, HID)), _rep((HID, HID)),
                _rep((1, HID)), _rep((HID, HID)), _rep((1, HID)),
                _rep((HID, 3))]

    def edge_weight_vals(w):
        return (w["w_rad"], w["wf_t"], w["w1_t"], w["b1"],
                w["wc0_t"], w["bc0"], w["wc1_t"])

    hcur, xcur = h0, x_pad
    ef = None
    for li in range(len(gcl)):
        w = gcl[li]
        hrx, hcx = gather2(tab_a, tab_b, rowp, colp)
        if li == 0:
            pay, ef = pl.pallas_call(
                _edge0_body,
                grid=(eg,),
                in_specs=[eb_full, eb_full,
                          pl.BlockSpec((EB, edge_attr.shape[1]), lambda i: (i, 0)),
                          _rep(fw2pi.shape), _rep((3 * HID, HID)), _rep((1, HID)),
                          _rep((1, HID)), _rep((1, HID)),
                          _rep((HID, HID)), _rep((HID, HID)), _rep((1, HID)),
                          _rep((HID, HID)), _rep((1, HID)),
                          _rep((HID, HID)), _rep((1, HID)),
                          _rep((edge_attr.shape[1], HID)), _rep((1, HID)),
                          _rep((HID, HID)), _rep((1, HID))]
                         + edge_weight_specs(),
                out_specs=[eb_pay, eb_feat],
                out_shape=[jax.ShapeDtypeStruct((EP, DGS), F32),
                           jax.ShapeDtypeStruct((EP, HID), F32)],
            )(hrx, hcx, ea_pad,
              fw2pi, wcm_t, bcm, w_sin, w_cos, wei_t, wej_t, bp0,
              wp1_t, bp1, wp2_t, bp2, we0_t, be0, we1_t, be1,
              *edge_weight_vals(w))
        else:
            pay = pl.pallas_call(
                _edge_mid_body,
                grid=(eg,),
                in_specs=[eb_full, eb_full, eb_feat] + edge_weight_specs(),
                out_specs=eb_pay,
                out_shape=jax.ShapeDtypeStruct((EP, DGS), F32),
            )(hrx, hcx, ef, *edge_weight_vals(w))

        accs = scatter(pay, rowp, zeros_acc)

        nu_common_specs = [
            pl.BlockSpec((BN, HID), lambda i: (i, 0)),
            pl.BlockSpec((BN, 16), lambda i: (i, 0)),
            pl.BlockSpec((BN, 16), lambda i: (i, 0)),
            pl.BlockSpec((2, BN, DA), lambda i: (0, i, 0)),
            node_w, node_w, node_b, node_w, node_b,
            node_w, node_b, _rep((HID, 1)), _rep((1, 1)),
        ]
        nu_common_vals = (hcur, xcur, velp, accs,
                          w["wn0a_t"], w["wn0b_t"], w["bn0"], w["wn1_t"],
                          w["bn1"], w["wv0_t"], w["bv0"], w["wv1_t"], w["bv1"])
        if li + 1 < len(gcl):
            wn = gcl[li + 1]
            hcur, xcur, tab_a, tab_b = pl.pallas_call(
                _node_update_mid_body,
                grid=(ng,),
                in_specs=nu_common_specs + [node_w, node_w, node_b],
                out_specs=[pl.BlockSpec((BN, HID), lambda i: (i, 0)),
                           pl.BlockSpec((BN, 16), lambda i: (i, 0)),
                           pl.BlockSpec((BN, DGT), lambda i: (i, 0)),
                           pl.BlockSpec((BN, DGT), lambda i: (i, 0))],
                out_shape=[jax.ShapeDtypeStruct((NP, HID), F32),
                           jax.ShapeDtypeStruct((NP, 16), F32),
                           jax.ShapeDtypeStruct((NP, DGT), F32),
                           jax.ShapeDtypeStruct((NP, DGT), F32)],
            )(*nu_common_vals, wn["wr_t"], wn["wc_t"], wn["b0"])
        else:
            hcur, xcur = pl.pallas_call(
                _node_update_last_body,
                grid=(ng,),
                in_specs=nu_common_specs,
                out_specs=[pl.BlockSpec((BN, HID), lambda i: (i, 0)),
                           pl.BlockSpec((BN, 16), lambda i: (i, 0))],
                out_shape=[jax.ShapeDtypeStruct((NP, HID), F32),
                           jax.ShapeDtypeStruct((NP, 16), F32)],
            )(*nu_common_vals)

    xf15 = xcur[:N, :3].reshape(G, 3 * NPTS)
    out15 = pl.pallas_call(
        _final_body,
        out_shape=jax.ShapeDtypeStruct((G, 3 * NPTS), F32),
    )(xf15, cent15)
    return out15.reshape(N, 3)
